# baseline (speedup 1.0000x reference)
import jax
import jax.numpy as jnp
from jax.experimental import pallas as pl
from jax.experimental.pallas import tpu as pltpu

BLOCK = 8192


def _mlp_block(x_ref, w1_ref, b1_ref, w2_ref, b2_ref, out_ref):
    h = jnp.tanh(
        jnp.dot(x_ref[...], w1_ref[...], preferred_element_type=jnp.float32)
        + b1_ref[...]
    )
    out_ref[...] = (
        jnp.dot(h, w2_ref[...], preferred_element_type=jnp.float32) + b2_ref[...]
    )


def kernel(x, W1, b1, W2, b2):
    n, d_in = x.shape
    hidden = W1.shape[0]
    d_out = W2.shape[0]
    w1t = W1.T
    w2t = W2.T
    b1r = b1.reshape(1, hidden)
    b2r = b2.reshape(1, d_out)

    grid = (n // BLOCK,)
    return pl.pallas_call(
        _mlp_block,
        grid=grid,
        in_specs=[
            pl.BlockSpec((BLOCK, d_in), lambda i: (i, 0)),
            pl.BlockSpec((d_in, hidden), lambda i: (0, 0)),
            pl.BlockSpec((1, hidden), lambda i: (0, 0)),
            pl.BlockSpec((hidden, d_out), lambda i: (0, 0)),
            pl.BlockSpec((1, d_out), lambda i: (0, 0)),
        ],
        out_specs=pl.BlockSpec((BLOCK, d_out), lambda i: (i, 0)),
        out_shape=jax.ShapeDtypeStruct((n, d_out), jnp.float32),
        compiler_params=pltpu.CompilerParams(
            dimension_semantics=("parallel",),
        ),
    )(x, w1t, b1r, w2t, b2r)


# speedup vs baseline: 1.0053x; 1.0053x over previous
import jax
import jax.numpy as jnp
from jax.experimental import pallas as pl
from jax.experimental.pallas import tpu as pltpu

BLOCK = 8192


def _mlp_block(x_ref, w1_ref, b1_ref, w2_ref, b2_ref, out_ref):
    xb = x_ref[...].astype(jnp.bfloat16)
    h = jnp.tanh(
        jnp.dot(xb, w1_ref[...], preferred_element_type=jnp.float32) + b1_ref[...]
    )
    out_ref[...] = (
        jnp.dot(
            h.astype(jnp.bfloat16), w2_ref[...], preferred_element_type=jnp.float32
        )
        + b2_ref[...]
    )


def kernel(x, W1, b1, W2, b2):
    n, d_in = x.shape
    hidden = W1.shape[0]
    d_out = W2.shape[0]
    w1t = W1.T.astype(jnp.bfloat16)
    w2t = W2.T.astype(jnp.bfloat16)
    b1r = b1.reshape(1, hidden)
    b2r = b2.reshape(1, d_out)

    grid = (n // BLOCK,)
    return pl.pallas_call(
        _mlp_block,
        grid=grid,
        in_specs=[
            pl.BlockSpec((BLOCK, d_in), lambda i: (i, 0)),
            pl.BlockSpec((d_in, hidden), lambda i: (0, 0)),
            pl.BlockSpec((1, hidden), lambda i: (0, 0)),
            pl.BlockSpec((hidden, d_out), lambda i: (0, 0)),
            pl.BlockSpec((1, d_out), lambda i: (0, 0)),
        ],
        out_specs=pl.BlockSpec((BLOCK, d_out), lambda i: (i, 0)),
        out_shape=jax.ShapeDtypeStruct((n, d_out), jnp.float32),
        compiler_params=pltpu.CompilerParams(
            dimension_semantics=("parallel",),
        ),
    )(x, w1t, b1r, w2t, b2r)
